# baseline (speedup 1.0000x reference)
import functools

import jax
import jax.numpy as jnp
from jax import lax
from jax.experimental import pallas as pl
from jax.experimental.pallas import tpu as pltpu
from jax.experimental.pallas import tpu_sc as plsc

_ROWS = 8
_N = 512
_LANES = 16
_CHUNKS = _N // _LANES

_MESH = plsc.VectorSubcoreMesh(core_axis_name="c", subcore_axis_name="s")


@functools.partial(
    pl.kernel,
    out_type=jax.ShapeDtypeStruct((_ROWS, _N), jnp.float32),
    mesh=_MESH,
    scratch_types=[
        pltpu.VMEM((_N,), jnp.float32),
        pltpu.VMEM((_N,), jnp.float32),
    ],
    compiler_params=pltpu.CompilerParams(needs_layout_passes=False),
)
def _sc_cumsum(d_hbm, out_hbm, row_v, acc_v):
    wid = lax.axis_index("s") * 2 + lax.axis_index("c")

    @pl.when(wid < _ROWS)
    def _():
        pltpu.sync_copy(d_hbm.at[wid], row_v)
        carry = jnp.float32(0.0)
        for i in range(_CHUNKS):
            v = row_v[pl.ds(i * _LANES, _LANES)]
            acc_v[pl.ds(i * _LANES, _LANES)] = plsc.cumsum(v) + carry
            carry = carry + jnp.sum(v)
        pltpu.sync_copy(acc_v, out_hbm.at[wid])


def kernel(durations, phoneme, frame):
    del phoneme, frame
    return _sc_cumsum(durations)


# speedup vs baseline: 1.0877x; 1.0877x over previous
import functools

import jax
import jax.numpy as jnp
from jax import lax
from jax.experimental import pallas as pl
from jax.experimental.pallas import tpu as pltpu
from jax.experimental.pallas import tpu_sc as plsc

_ROWS = 8
_N = 512
_LANES = 16
_CHUNKS = _N // _LANES

_MESH = plsc.VectorSubcoreMesh(
    core_axis_name="c", subcore_axis_name="s", num_cores=1
)


@functools.partial(
    pl.kernel,
    out_type=jax.ShapeDtypeStruct((_ROWS, _N), jnp.float32),
    mesh=_MESH,
    scratch_types=[
        pltpu.VMEM((_N,), jnp.float32),
        pltpu.VMEM((_N,), jnp.float32),
    ],
    compiler_params=pltpu.CompilerParams(needs_layout_passes=False),
)
def _sc_cumsum(d_hbm, out_hbm, row_v, acc_v):
    wid = lax.axis_index("s")

    @pl.when(wid < _ROWS)
    def _():
        pltpu.sync_copy(d_hbm.at[wid], row_v)
        carry = jnp.float32(0.0)
        for i in range(_CHUNKS):
            v = row_v[pl.ds(i * _LANES, _LANES)]
            acc_v[pl.ds(i * _LANES, _LANES)] = plsc.cumsum(v) + carry
            carry = carry + jnp.sum(v)
        pltpu.sync_copy(acc_v, out_hbm.at[wid])


def kernel(durations, phoneme, frame):
    del phoneme, frame
    return _sc_cumsum(durations)


# speedup vs baseline: 12.7089x; 11.6845x over previous
import jax
import jax.numpy as jnp
from jax import lax
from jax.experimental import pallas as pl


def _cumsum_kernel(d_ref, o_ref):
    x = d_ref[...]
    n = x.shape[1]
    row = lax.broadcasted_iota(jnp.int32, (n, n), 0)
    col = lax.broadcasted_iota(jnp.int32, (n, n), 1)
    tri = jnp.where(row <= col, 1.0, 0.0).astype(x.dtype)
    o_ref[...] = jnp.dot(x, tri, preferred_element_type=jnp.float32)


def kernel(durations, phoneme, frame):
    del phoneme, frame
    return pl.pallas_call(
        _cumsum_kernel,
        out_shape=jax.ShapeDtypeStruct(durations.shape, durations.dtype),
    )(durations)


# speedup vs baseline: 12.7297x; 1.0016x over previous
import jax
import jax.numpy as jnp
from jax import lax
from jax.experimental import pallas as pl


def _cumsum_kernel(d_ref, o_ref):
    x = d_ref[...]
    n = x.shape[1]
    row = lax.broadcasted_iota(jnp.int32, (n, n), 0)
    col = lax.broadcasted_iota(jnp.int32, (n, n), 1)
    tri = jnp.where(row <= col, 1.0, 0.0).astype(jnp.bfloat16)
    o_ref[...] = jnp.dot(
        x.astype(jnp.bfloat16), tri, preferred_element_type=jnp.float32
    )


def kernel(durations, phoneme, frame):
    del phoneme, frame
    return pl.pallas_call(
        _cumsum_kernel,
        out_shape=jax.ShapeDtypeStruct(durations.shape, durations.dtype),
    )(durations)


# speedup vs baseline: 12.7745x; 1.0035x over previous
import jax
import jax.numpy as jnp
from jax import lax
from jax.experimental import pallas as pl


def _cumsum_kernel(d_ref, o_ref):
    x = d_ref[...]
    n = x.shape[1]
    row = lax.broadcasted_iota(jnp.int32, (n, n), 0)
    col = lax.broadcasted_iota(jnp.int32, (n, n), 1)
    tri = jnp.where(row <= col, 1.0, 0.0).astype(x.dtype)
    o_ref[...] = jnp.dot(x, tri, preferred_element_type=jnp.float32)


def kernel(durations, phoneme, frame):
    del phoneme, frame
    return pl.pallas_call(
        _cumsum_kernel,
        out_shape=jax.ShapeDtypeStruct(durations.shape, durations.dtype),
    )(durations)
